# Initial kernel scaffold; baseline (speedup 1.0000x reference)
import jax
import jax.numpy as jnp
from jax.experimental import pallas as pl


def kernel(p, t, control_points, W0, W1, W2, W3, W4, W5):
    raise NotImplementedError("write your pallas kernel here")


# speedup vs baseline: 27.9180x; 27.9180x over previous
import numpy as np
import jax
import jax.numpy as jnp
from jax.experimental import pallas as pl

_NUM_VN = 6
_INV2T2 = 50.0
_BIG = 3.0e38


def _transforms_kernel(x4_ref, w0, w1, w2, w3, w4, w5, out_ref):
    x = x4_ref[...]
    feats = [x]
    for f in (1.0, 2.0):
        feats.append(jnp.sin(x * f))
        feats.append(jnp.cos(x * f))
    h = jnp.concatenate(feats, axis=-1)
    for w in (w0, w1, w2, w3, w4):
        h = jnp.maximum(jnp.dot(h, w[...], preferred_element_type=jnp.float32), 0.0)
    o = jnp.dot(h, w5[...], preferred_element_type=jnp.float32)
    ax, ay, az = o[:, 0:1], o[:, 1:2], o[:, 2:3]
    x2, y2, z2 = ax * ax, ay * ay, az * az
    theta2 = x2 + y2 + z2
    theta = jnp.sqrt(theta2 + 1e-12)
    A = jnp.sin(theta) / theta
    B = (1.0 - jnp.cos(theta)) / (theta2 + 1e-12)
    xy, xz, yz = ax * ay, ax * az, ay * az
    r00 = 1.0 - B * (y2 + z2)
    r01 = -A * az + B * xy
    r02 = A * ay + B * xz
    r10 = A * az + B * xy
    r11 = 1.0 - B * (x2 + z2)
    r12 = -A * ax + B * yz
    r20 = -A * ay + B * xz
    r21 = A * ax + B * yz
    r22 = 1.0 - B * (x2 + y2)
    out_ref[...] = jnp.concatenate(
        [r00, r01, r02, r10, r11, r12, r20, r21, r22,
         o[:, 3:4], o[:, 4:5], o[:, 5:6]], axis=-1)


def _knn_blend_kernel(pp_ref, c_ref, tab_ref, gx_ref, gy_ref, gz_ref, gt_ref,
                      out_ref):
    pp = pp_ref[...]
    e = jnp.dot(pp, c_ref[...], preferred_element_type=jnp.float32)
    ew = e
    m1 = None
    mk = None
    for k in range(_NUM_VN):
        mk = jnp.min(ew, axis=1, keepdims=True)
        if k == 0:
            m1 = mk
        if k < _NUM_VN - 1:
            ew = jnp.where(ew <= mk, _BIG, ew)
    s = jnp.where(e <= mk, jnp.exp((m1 - e) * _INV2T2), 0.0)
    z = jnp.sum(s, axis=1, keepdims=True)
    s = s / z
    blended = jnp.dot(s, tab_ref[...], preferred_element_type=jnp.float32)
    px, py, pz = pp[:, 0:1], pp[:, 1:2], pp[:, 2:3]
    out_ref[...] = (
        jnp.dot(blended, gx_ref[...], preferred_element_type=jnp.float32) * px
        + jnp.dot(blended, gy_ref[...], preferred_element_type=jnp.float32) * py
        + jnp.dot(blended, gz_ref[...], preferred_element_type=jnp.float32) * pz
        + jnp.dot(blended, gt_ref[...], preferred_element_type=jnp.float32))


def kernel(p, t, control_points, W0, W1, W2, W3, W4, W5):
    n = p.shape[0]
    b = t.shape[0]
    ncp = control_points.shape[0]

    tcol = jnp.repeat(t, ncp)[:, None]
    cps = jnp.tile(control_points, (b, 1))
    x4 = jnp.concatenate([tcol, cps], axis=1)
    R = 1000
    tab12 = pl.pallas_call(
        _transforms_kernel,
        grid=(b * ncp // R,),
        in_specs=[
            pl.BlockSpec((R, 4), lambda i: (i, 0)),
            pl.BlockSpec(W0.shape, lambda i: (0, 0)),
            pl.BlockSpec(W1.shape, lambda i: (0, 0)),
            pl.BlockSpec(W2.shape, lambda i: (0, 0)),
            pl.BlockSpec(W3.shape, lambda i: (0, 0)),
            pl.BlockSpec(W4.shape, lambda i: (0, 0)),
            pl.BlockSpec(W5.shape, lambda i: (0, 0)),
        ],
        out_specs=pl.BlockSpec((R, 12), lambda i: (i, 0)),
        out_shape=jax.ShapeDtypeStruct((b * ncp, 12), jnp.float32),
    )(x4, W0, W1, W2, W3, W4, W5)
    table = tab12.reshape(b, ncp, 12).transpose(1, 0, 2).reshape(ncp, b * 12)

    pp = jnp.concatenate(
        [p, jnp.ones((n, 1), jnp.float32), jnp.zeros((n, 4), jnp.float32)],
        axis=1)
    csq = jnp.sum(control_points * control_points, axis=1)
    C = jnp.concatenate(
        [-2.0 * control_points.T, csq[None, :],
         jnp.zeros((4, ncp), jnp.float32)], axis=0)

    G = np.zeros((4, b * 12, b * 3), np.float32)
    for bb in range(b):
        for c in range(3):
            for dd in range(3):
                G[dd, bb * 12 + 3 * c + dd, bb * 3 + c] = 1.0
            G[3, bb * 12 + 9 + c, bb * 3 + c] = 1.0
    gx, gy, gz, gt = (jnp.asarray(G[i]) for i in range(4))

    P = 1000
    out24 = pl.pallas_call(
        _knn_blend_kernel,
        grid=(n // P,),
        in_specs=[
            pl.BlockSpec((P, 8), lambda i: (i, 0)),
            pl.BlockSpec((8, ncp), lambda i: (0, 0)),
            pl.BlockSpec((ncp, b * 12), lambda i: (0, 0)),
            pl.BlockSpec((b * 12, b * 3), lambda i: (0, 0)),
            pl.BlockSpec((b * 12, b * 3), lambda i: (0, 0)),
            pl.BlockSpec((b * 12, b * 3), lambda i: (0, 0)),
            pl.BlockSpec((b * 12, b * 3), lambda i: (0, 0)),
        ],
        out_specs=pl.BlockSpec((P, b * 3), lambda i: (i, 0)),
        out_shape=jax.ShapeDtypeStruct((n, b * 3), jnp.float32),
    )(pp, C, table, gx, gy, gz, gt)
    return out24.reshape(n, b, 3).transpose(1, 0, 2)
